# baseline (speedup 1.0000x reference)
import functools

import jax
import jax.numpy as jnp
import numpy as np
from jax import lax
from jax.experimental import pallas as pl
from jax.experimental.pallas import tpu as pltpu
from jax.experimental.pallas import tpu_sc as plsc

NUM_DIM = 64
LANES = 16
NC, NS = 2, 16
NW = NC * NS
CHUNK = 128

_BITREV = [0, 8, 4, 12, 2, 10, 6, 14, 1, 9, 5, 13, 3, 11, 7, 15]
_LANE = np.arange(LANES, dtype=np.int32)


def _irt_body(theta_hbm, alpha_hbm, beta_hbm, sid_hbm, qid_hbm, out_hbm,
              sidx_v, qidx_v, theta_v, alpha_v, beta_v, out_v, sem,
              *, rows_per_worker, n_chunks):
    wid = lax.axis_index("s") * NC + lax.axis_index("c")

    pltpu.sync_copy(sid_hbm.at[wid], sidx_v)
    pltpu.sync_copy(qid_hbm.at[wid], qidx_v)

    descs = []
    for j in range(n_chunks):
        sl = pl.ds(j * CHUNK, CHUNK)
        descs.append(pltpu.make_async_copy(
            theta_hbm.at[sidx_v.at[j]], theta_v.at[sl], sem))
        descs.append(pltpu.make_async_copy(
            alpha_hbm.at[qidx_v.at[j]], alpha_v.at[sl], sem))
        descs.append(pltpu.make_async_copy(
            beta_hbm.at[qidx_v.at[j]], beta_v.at[sl], sem))
    for d in descs:
        d.start()
    for d in descs:
        d.wait()

    lane = lax.iota(jnp.int32, LANES)
    perm_idx = {d: lane ^ d for d in (8, 4, 2, 1)}
    masks = {d: (lane & d) == 0 for d in (8, 4, 2, 1)}

    gather_dnums = lax.GatherDimensionNumbers(
        offset_dims=(), collapsed_slice_dims=(0,), start_index_map=(0,))

    def permute(v, d):
        return lax.gather(v, perm_idx[d][:, None], gather_dnums,
                          slice_sizes=(1,),
                          mode=lax.GatherScatterMode.PROMISE_IN_BOUNDS)

    def group(g, _):
        base = g * LANES
        vs = []
        for k in range(LANES):
            r = base + _BITREV[k]
            acc = None
            for c in range(NUM_DIM // LANES):
                sl = pl.ds(c * LANES, LANES)
                p = alpha_v[r, sl] * (theta_v[r, sl] - beta_v[r, sl])
                acc = p if acc is None else acc + p
            vs.append(acc)
        for d in (8, 4, 2, 1):
            vs = [jnp.where(masks[d], a + permute(a, d), b + permute(b, d))
                  for a, b in zip(vs[0::2], vs[1::2])]
        res = vs[0]
        out_v[pl.ds(base, LANES)] = 1.0 / (1.0 + jnp.exp(-res))
        return ()

    lax.fori_loop(0, rows_per_worker // LANES, group, ())

    pltpu.sync_copy(out_v, out_hbm.at[pl.ds(wid * rows_per_worker,
                                            rows_per_worker)])


@jax.jit
def kernel(student_ids, question_ids, theta_table, alpha_table, beta_table):
    batch = student_ids.shape[0]
    rows_per_worker = batch // NW
    n_chunks = rows_per_worker // CHUNK

    sid = student_ids.astype(jnp.int32).reshape(NW, n_chunks, CHUNK)
    qid = question_ids.astype(jnp.int32).reshape(NW, n_chunks, CHUNK)

    mesh = plsc.VectorSubcoreMesh(core_axis_name="c", subcore_axis_name="s")
    body = functools.partial(_irt_body, rows_per_worker=rows_per_worker,
                             n_chunks=n_chunks)
    run = pl.kernel(
        body,
        mesh=mesh,
        compiler_params=pltpu.CompilerParams(use_tc_tiling_on_sc=False),
        out_type=jax.ShapeDtypeStruct((batch,), jnp.float32),
        scratch_types=[
            pltpu.VMEM((n_chunks, CHUNK), jnp.int32),
            pltpu.VMEM((n_chunks, CHUNK), jnp.int32),
            pltpu.VMEM((rows_per_worker, NUM_DIM), jnp.float32),
            pltpu.VMEM((rows_per_worker, NUM_DIM), jnp.float32),
            pltpu.VMEM((rows_per_worker, NUM_DIM), jnp.float32),
            pltpu.VMEM((rows_per_worker,), jnp.float32),
            pltpu.SemaphoreType.DMA,
        ],
    )
    pred = run(theta_table, alpha_table, beta_table, sid, qid)
    return pred.reshape(batch, 1)
